# baseline (speedup 1.0000x reference)
import functools

import jax
import jax.numpy as jnp
from jax import lax
from jax.experimental import pallas as pl
from jax.experimental.pallas import tpu as pltpu
from jax.experimental.pallas import tpu_sc as plsc

_LN_EPS = 1e-5
_TOKEN_BLOCK = 1024

_SC_CORES = 2
_SC_SUBCORES = 16
_SC_WORKERS = _SC_CORES * _SC_SUBCORES
_LANES = 16
_SC_CHUNK = 512


def _mlp_body(x_ref, w1_ref, b1_ref, g_ref, be_ref, w2_ref, b2_ref, it_ref,
              logits_ref, logits_t_ref):
    x = x_ref[...]
    h = jnp.dot(x, w1_ref[...], preferred_element_type=jnp.float32)
    h = h + b1_ref[...]
    mu = jnp.mean(h, axis=-1, keepdims=True)
    var = jnp.mean((h - mu) ** 2, axis=-1, keepdims=True)
    h = (h - mu) / jnp.sqrt(var + _LN_EPS) * g_ref[...] + be_ref[...]
    h = 0.5 * h * (1.0 + jax.lax.erf(h * 0.7071067811865476))
    logits = jnp.dot(h, w2_ref[...], preferred_element_type=jnp.float32)
    logits = (logits + b2_ref[...]) * it_ref[0, 0]
    logits_ref[...] = logits
    logits_t_ref[...] = logits.T


def _mlp_logits(x, W1, b1, ln_gamma, ln_beta, W2, b2, temperature):
    n, d = x.shape
    h = W1.shape[1]
    e = W2.shape[1]
    bt = min(_TOKEN_BLOCK, n)
    inv_t = (1.0 / jnp.clip(temperature, 0.5, 5.0)).reshape(1, 1)

    return pl.pallas_call(
        _mlp_body,
        grid=(n // bt,),
        in_specs=[
            pl.BlockSpec((bt, d), lambda i: (i, 0)),
            pl.BlockSpec((d, h), lambda i: (0, 0)),
            pl.BlockSpec((1, h), lambda i: (0, 0)),
            pl.BlockSpec((1, h), lambda i: (0, 0)),
            pl.BlockSpec((1, h), lambda i: (0, 0)),
            pl.BlockSpec((h, e), lambda i: (0, 0)),
            pl.BlockSpec((1, e), lambda i: (0, 0)),
            pl.BlockSpec((1, 1), lambda i: (0, 0)),
        ],
        out_specs=(
            pl.BlockSpec((bt, e), lambda i: (i, 0)),
            pl.BlockSpec((e, bt), lambda i: (0, i)),
        ),
        out_shape=(
            jax.ShapeDtypeStruct((n, e), jnp.float32),
            jax.ShapeDtypeStruct((e, n), jnp.float32),
        ),
        compiler_params=pltpu.CompilerParams(
            dimension_semantics=("arbitrary",),
        ),
    )(x, W1, b1.reshape(1, h), ln_gamma.reshape(1, h), ln_beta.reshape(1, h),
      W2, b2.reshape(1, e), inv_t)


def _routing_sc(logits_t):
    e, n = logits_t.shape
    tpw = n // _SC_WORKERS
    nchunks = tpw // _SC_CHUNK
    groups = _SC_CHUNK // _LANES
    mesh = plsc.VectorSubcoreMesh(core_axis_name="c", subcore_axis_name="s")

    @functools.partial(
        pl.kernel,
        mesh=mesh,
        out_type=(
            jax.ShapeDtypeStruct((n * e,), jnp.float32),
            jax.ShapeDtypeStruct((n * 2,), jnp.int32),
        ),
        scratch_types=[
            pltpu.VMEM((e, _SC_CHUNK), jnp.float32),
            pltpu.VMEM((_SC_CHUNK * e,), jnp.float32),
            pltpu.VMEM((_SC_CHUNK * 2,), jnp.int32),
        ],
        compiler_params=pltpu.CompilerParams(needs_layout_passes=False),
    )
    def route(lt_hbm, gates_hbm, idx_hbm, lt_v, gates_v, idx_v):
        wid = lax.axis_index("s") * _SC_CORES + lax.axis_index("c")
        base = wid * tpw
        iota = lax.iota(jnp.int32, _LANES)
        zero_row = jnp.zeros((_LANES,), jnp.float32)
        col0 = jnp.zeros((_LANES,), jnp.int32)
        col1 = col0 + 1
        neg_inf = jnp.full((_LANES,), -jnp.inf, jnp.float32)

        def do_chunk(c, carry):
            off = base + c * _SC_CHUNK
            pltpu.sync_copy(lt_hbm.at[:, pl.ds(off, _SC_CHUNK)], lt_v)

            def do_group(g, carry2):
                go = g * _LANES
                for q in range(_LANES * e // _LANES):
                    gates_v[pl.ds(go * e + q * _LANES, _LANES)] = zero_row
                m0 = lt_v[0, pl.ds(go, _LANES)]
                i0 = col0
                m1 = neg_inf
                i1 = col0
                for ee in range(1, e):
                    v = lt_v[ee, pl.ds(go, _LANES)]
                    gt0 = v > m0
                    gt1 = v > m1
                    m1 = jnp.where(gt0, m0, jnp.where(gt1, v, m1))
                    i1 = jnp.where(gt0, i0, jnp.where(gt1, ee, i1))
                    m0 = jnp.where(gt0, v, m0)
                    i0 = jnp.where(gt0, ee, i0)
                e1 = jnp.exp(m1 - m0)
                scale = 1.0 / ((1.0 + e1) * (1.0 + 1e-10))
                g1 = e1 * scale
                rows = go + iota
                plsc.store_scatter(gates_v, [rows * e + i0], scale)
                plsc.store_scatter(gates_v, [rows * e + i1], g1)
                plsc.store_scatter(idx_v, [rows * 2 + col0], i0)
                plsc.store_scatter(idx_v, [rows * 2 + col1], i1)
                return carry2

            lax.fori_loop(0, groups, do_group, 0, unroll=False)
            pltpu.sync_copy(gates_v, gates_hbm.at[pl.ds(off * e, _SC_CHUNK * e)])
            pltpu.sync_copy(idx_v, idx_hbm.at[pl.ds(off * 2, _SC_CHUNK * 2)])
            return carry

        lax.fori_loop(0, nchunks, do_chunk, 0, unroll=False)

    gates_flat, idx_flat = route(logits_t)
    return gates_flat.reshape(n, e), idx_flat.reshape(n, 2)


def kernel(x, W1, b1, ln_gamma, ln_beta, W2, b2, temperature):
    logits, logits_t = _mlp_logits(x, W1, b1, ln_gamma, ln_beta, W2, b2,
                                   temperature)
    gates, idx = _routing_sc(logits_t)
    return (gates, idx, logits)


# speedup vs baseline: 1.0278x; 1.0278x over previous
import functools

import jax
import jax.numpy as jnp
from jax import lax
from jax.experimental import pallas as pl
from jax.experimental.pallas import tpu as pltpu
from jax.experimental.pallas import tpu_sc as plsc

_LN_EPS = 1e-5
_TOKEN_BLOCK = 1024

_SC_CORES = 2
_SC_SUBCORES = 16
_SC_WORKERS = _SC_CORES * _SC_SUBCORES
_LANES = 16
_SC_CHUNK = 512


def _mlp_body(x_ref, w1_ref, b1_ref, g_ref, be_ref, w2_ref, b2_ref, it_ref,
              logits_ref, logits_t_ref):
    x = x_ref[...]
    h = jnp.dot(x, w1_ref[...], preferred_element_type=jnp.float32)
    h = h + b1_ref[...]
    mu = jnp.mean(h, axis=-1, keepdims=True)
    var = jnp.mean((h - mu) ** 2, axis=-1, keepdims=True)
    h = (h - mu) / jnp.sqrt(var + _LN_EPS) * g_ref[...] + be_ref[...]
    h = 0.5 * h * (1.0 + jax.lax.erf(h * 0.7071067811865476))
    logits = jnp.dot(h, w2_ref[...], preferred_element_type=jnp.float32)
    logits = (logits + b2_ref[...]) * it_ref[0, 0]
    logits_ref[...] = logits
    logits_t_ref[...] = logits.T


def _mlp_logits(x, W1, b1, ln_gamma, ln_beta, W2, b2, temperature):
    n, d = x.shape
    h = W1.shape[1]
    e = W2.shape[1]
    bt = min(_TOKEN_BLOCK, n)
    inv_t = (1.0 / jnp.clip(temperature, 0.5, 5.0)).reshape(1, 1)

    return pl.pallas_call(
        _mlp_body,
        grid=(n // bt,),
        in_specs=[
            pl.BlockSpec((bt, d), lambda i: (i, 0)),
            pl.BlockSpec((d, h), lambda i: (0, 0)),
            pl.BlockSpec((1, h), lambda i: (0, 0)),
            pl.BlockSpec((1, h), lambda i: (0, 0)),
            pl.BlockSpec((1, h), lambda i: (0, 0)),
            pl.BlockSpec((h, e), lambda i: (0, 0)),
            pl.BlockSpec((1, e), lambda i: (0, 0)),
            pl.BlockSpec((1, 1), lambda i: (0, 0)),
        ],
        out_specs=(
            pl.BlockSpec((bt, e), lambda i: (i, 0)),
            pl.BlockSpec((e, bt), lambda i: (0, i)),
        ),
        out_shape=(
            jax.ShapeDtypeStruct((n, e), jnp.float32),
            jax.ShapeDtypeStruct((e, n), jnp.float32),
        ),
        compiler_params=pltpu.CompilerParams(
            dimension_semantics=("arbitrary",),
        ),
    )(x, W1, b1.reshape(1, h), ln_gamma.reshape(1, h), ln_beta.reshape(1, h),
      W2, b2.reshape(1, e), inv_t)


def _routing_sc(logits_t):
    e, n = logits_t.shape
    tpw = n // _SC_WORKERS
    nchunks = tpw // _SC_CHUNK
    groups = _SC_CHUNK // _LANES
    mesh = plsc.VectorSubcoreMesh(core_axis_name="c", subcore_axis_name="s")

    @functools.partial(
        pl.kernel,
        mesh=mesh,
        out_type=(
            jax.ShapeDtypeStruct((n, e), jnp.float32),
            jax.ShapeDtypeStruct((n * 2,), jnp.int32),
        ),
        scratch_types=[
            pltpu.VMEM((e, _SC_CHUNK), jnp.float32),
            pltpu.VMEM((_SC_CHUNK, e), jnp.float32),
            pltpu.VMEM((_SC_CHUNK * 2,), jnp.int32),
        ],
        compiler_params=pltpu.CompilerParams(needs_layout_passes=False),
    )
    def route(lt_hbm, gates_hbm, idx_hbm, lt_v, gates_v, idx_v):
        wid = lax.axis_index("s") * _SC_CORES + lax.axis_index("c")
        base = wid * tpw
        iota = lax.iota(jnp.int32, _LANES)
        zero_row = jnp.zeros((_LANES,), jnp.float32)
        col0 = jnp.zeros((_LANES,), jnp.int32)
        col1 = col0 + 1
        neg_inf = jnp.full((_LANES,), -jnp.inf, jnp.float32)

        def do_chunk(c, carry):
            off = base + c * _SC_CHUNK
            pltpu.sync_copy(lt_hbm.at[:, pl.ds(off, _SC_CHUNK)], lt_v)

            def do_group(g, carry2):
                go = g * _LANES
                for r in range(_LANES):
                    for q in range(e // _LANES):
                        gates_v[go + r, pl.ds(q * _LANES, _LANES)] = zero_row
                m0 = lt_v[0, pl.ds(go, _LANES)]
                i0 = col0
                m1 = neg_inf
                i1 = col0
                for ee in range(1, e):
                    v = lt_v[ee, pl.ds(go, _LANES)]
                    gt0 = v > m0
                    gt1 = v > m1
                    m1 = jnp.where(gt0, m0, jnp.where(gt1, v, m1))
                    i1 = jnp.where(gt0, i0, jnp.where(gt1, ee, i1))
                    m0 = jnp.where(gt0, v, m0)
                    i0 = jnp.where(gt0, ee, i0)
                e1 = jnp.exp(m1 - m0)
                scale = 1.0 / ((1.0 + e1) * (1.0 + 1e-10))
                g1 = e1 * scale
                rows = go + iota
                plsc.store_scatter(gates_v, [rows, i0], scale)
                plsc.store_scatter(gates_v, [rows, i1], g1)
                plsc.store_scatter(idx_v, [rows * 2 + col0], i0)
                plsc.store_scatter(idx_v, [rows * 2 + col1], i1)
                return carry2

            lax.fori_loop(0, groups, do_group, 0, unroll=False)
            pltpu.sync_copy(gates_v, gates_hbm.at[pl.ds(off, _SC_CHUNK), :])
            pltpu.sync_copy(idx_v, idx_hbm.at[pl.ds(off * 2, _SC_CHUNK * 2)])
            return carry

        lax.fori_loop(0, nchunks, do_chunk, 0, unroll=False)

    gates, idx_flat = route(logits_t)
    return gates, idx_flat.reshape(n, 2)


def kernel(x, W1, b1, ln_gamma, ln_beta, W2, b2, temperature):
    logits, logits_t = _mlp_logits(x, W1, b1, ln_gamma, ln_beta, W2, b2,
                                   temperature)
    gates, idx = _routing_sc(logits_t)
    return (gates, idx, logits)
